# baseline (speedup 1.0000x reference)
import functools

import jax
import jax.numpy as jnp
from jax.experimental import pallas as pl
from jax.experimental.pallas import tpu as pltpu

B, N, DIM, DEPTH, K, HID, NC = 4, 2048, 64, 2, 16, 128, 13
EPS = 1e-5


def _bn1d(x, g, b):
    m = x.mean(axis=(0, 2), keepdims=True)
    v = x.var(axis=(0, 2), keepdims=True)
    return (x - m) / jnp.sqrt(v + EPS) * g[None, :, None] + b[None, :, None]


def _bn2d(x, g, b):
    m = x.mean(axis=(0, 2, 3), keepdims=True)
    v = x.var(axis=(0, 2, 3), keepdims=True)
    return (x - m) / jnp.sqrt(v + EPS) * g[None, :, None, None] + b[None, :, None, None]


def _c1(W, x):
    return jnp.einsum('oi,bin->bon', W, x)


def _c2(W, x):
    return jnp.einsum('oi,bink->bonk', W, x)


def _head_kernel(x_ref, ws1_ref, gs_ref, bs_ref, ws2_ref, bs2_ref, out_ref):
    ws1 = ws1_ref[...]
    ws2 = ws2_ref[...]
    hs = []
    s = jnp.zeros((HID, 1), jnp.float32)
    sq = jnp.zeros((HID, 1), jnp.float32)
    for b in range(B):
        h = jax.lax.dot(ws1, x_ref[b], preferred_element_type=jnp.float32)
        hs.append(h)
        s = s + jnp.sum(h, axis=1, keepdims=True)
        sq = sq + jnp.sum(h * h, axis=1, keepdims=True)
    cnt = float(B * N)
    m = s / cnt
    v = sq / cnt - m * m
    scale = gs_ref[...].reshape(HID, 1) / jnp.sqrt(v + EPS)
    shift = bs_ref[...].reshape(HID, 1) - m * scale
    for b in range(B):
        h = jax.nn.relu(hs[b] * scale + shift)
        out_ref[b] = jax.lax.dot(ws2, h, preferred_element_type=jnp.float32) \
            + bs2_ref[...].reshape(NC, 1)


def _head(x, Ws1, g_s, b_s, Ws2, bs2):
    return pl.pallas_call(
        _head_kernel,
        out_shape=jax.ShapeDtypeStruct((B, NC, N), jnp.float32),
    )(x, Ws1, g_s, b_s, Ws2, bs2)


def kernel(points, We, g_e, b_e, Wq, Wk, Wv, Wpe1, g_pe, b_pe, Wpe2, Wam1, g_am, b_am, Wam2, g1, b1, Wf1, g_f, b_f, Wf2, g2, b2, Ws1, g_s, b_s, Ws2, bs2):
    Bb, Nn, _ = points.shape
    xyz = jnp.transpose(points, (0, 2, 1))
    x = jax.nn.relu(_bn1d(_c1(We, xyz), g_e, b_e))
    sq = jnp.sum(points * points, axis=-1)
    d2 = sq[:, :, None] + sq[:, None, :] - 2.0 * jnp.einsum('bnd,bmd->bnm', points, points)
    d2 = jax.lax.stop_gradient(d2)
    _, idx = jax.lax.top_k(-d2, K)
    idx_flat = idx.reshape(Bb, Nn * K)
    for i in range(DEPTH):
        D = x.shape[1]
        q = _c1(Wq[i], x)
        k_ = _c1(Wk[i], x)
        v_ = _c1(Wv[i], x)
        k_nn = jnp.take_along_axis(k_, idx_flat[:, None, :], axis=2).reshape(Bb, D, Nn, K)
        v_nn = jnp.take_along_axis(v_, idx_flat[:, None, :], axis=2).reshape(Bb, D, Nn, K)
        xyz_nn = jnp.take_along_axis(xyz, idx_flat[:, None, :], axis=2).reshape(Bb, 3, Nn, K)
        pos_diff = xyz[:, :, :, None] - xyz_nn
        pe = _c2(Wpe2[i], jax.nn.relu(_bn2d(_c2(Wpe1[i], pos_diff), g_pe[i], b_pe[i])))
        attn = _c2(Wam2[i], jax.nn.relu(_bn2d(_c2(Wam1[i], q[:, :, :, None] - k_nn + pe), g_am[i], b_am[i])))
        attn = jax.nn.softmax(attn, axis=-1)
        out = jnp.sum(attn * (v_nn + pe), axis=-1)
        x = _bn1d(x + out, g1[i], b1[i])
        ffn = _c1(Wf2[i], jax.nn.relu(_bn1d(_c1(Wf1[i], x), g_f[i], b_f[i])))
        x = _bn1d(x + ffn, g2[i], b2[i])
    return _head(x, Ws1, g_s, b_s, Ws2, bs2)


# speedup vs baseline: 10.2073x; 10.2073x over previous
import functools

import jax
import jax.numpy as jnp
from jax import lax
from jax.experimental import pallas as pl
from jax.experimental.pallas import tpu as pltpu
from jax.experimental.pallas import tpu_sc as plsc

B, N, DIM, DEPTH, K, HID, NC = 4, 2048, 64, 2, 16, 128, 13
EPS = 1e-5
RB = 256
NB = 256
NBLK = N // NB
NBK = NB * K


def _knn_body(p_ref, pt_ref, o_ref):
    p = p_ref[0]
    pt = pt_ref[0]
    sqr = jnp.sum(p * p, axis=1, keepdims=True)
    sqa = jnp.sum(pt * pt, axis=0, keepdims=True)
    dot = lax.dot_general(p.astype(jnp.bfloat16), pt.astype(jnp.bfloat16),
                          (((1,), (0,)), ((), ())),
                          preferred_element_type=jnp.float32)
    d = (sqr + sqa) - 2.0 * dot
    cols = lax.broadcasted_iota(jnp.int32, (RB, N), 1)
    kcols = lax.broadcasted_iota(jnp.int32, (RB, K), 1)
    kidx = jnp.zeros((RB, K), jnp.int32)
    for j in range(K):
        m = jnp.min(d, axis=1, keepdims=True)
        am = jnp.min(jnp.where(d == m, cols, N), axis=1, keepdims=True)
        kidx = jnp.where(kcols == j, am, kidx)
        d = jnp.where(cols == am, jnp.float32(jnp.inf), d)
    o_ref[0] = kidx + pl.program_id(0) * N


def _knn(points, pT):
    return pl.pallas_call(
        _knn_body,
        grid=(B, N // RB),
        in_specs=[
            pl.BlockSpec((1, RB, 3), lambda b, r: (b, r, 0)),
            pl.BlockSpec((1, 3, N), lambda b, r: (b, 0, 0)),
        ],
        out_specs=pl.BlockSpec((1, RB, K), lambda b, r: (b, r, 0)),
        out_shape=jax.ShapeDtypeStruct((B, N, K), jnp.int32),
    )(points, pT)


def _make_sc_gather(R, D, M):
    NW = 32
    Mw = M // NW
    CH = min(Mw, 65536 // D)
    NCH = Mw // CH
    mesh = plsc.VectorSubcoreMesh(core_axis_name="c", subcore_axis_name="s")

    @functools.partial(
        pl.kernel,
        mesh=mesh,
        compiler_params=pltpu.CompilerParams(use_tc_tiling_on_sc=False),
        out_type=jax.ShapeDtypeStruct((M, D), jnp.float32),
        scratch_types=[
            pltpu.VMEM((CH,), jnp.int32),
            pltpu.VMEM((CH, D), jnp.float32),
            pltpu.SemaphoreType.DMA,
        ],
    )
    def g(tab_hbm, idx_hbm, out_hbm, idx_v, rows_v, sem):
        wid = lax.axis_index("s") * 2 + lax.axis_index("c")
        base = wid * Mw
        for c in range(NCH):
            off = base + c * CH
            pltpu.sync_copy(idx_hbm.at[pl.ds(off, CH)], idx_v)
            pltpu.async_copy(tab_hbm.at[idx_v], rows_v, sem).wait()
            pltpu.sync_copy(rows_v, out_hbm.at[pl.ds(off, CH)])

    return g


def _acc_stats(acc_ref, st_ref, s, sq, nblk_last):
    first = (pl.program_id(0) == 0) & (pl.program_id(1) == 0)
    last = (pl.program_id(0) == B - 1) & (pl.program_id(1) == nblk_last)

    @pl.when(first)
    def _():
        acc_ref[...] = jnp.zeros_like(acc_ref)

    c = s.shape[-1]
    acc_ref[0:1] = acc_ref[0:1] + s.reshape(1, c)
    acc_ref[1:2] = acc_ref[1:2] + sq.reshape(1, c)

    @pl.when(last)
    def _():
        st_ref[...] = acc_ref[...]


def _embed_body(p_ref, w_ref, e_ref, st_ref, acc_ref):
    p = p_ref[0]
    w = w_ref[...]
    e = p[:, 0:1] * w[0:1, :]
    e = e + p[:, 1:2] * w[1:2, :]
    e = e + p[:, 2:3] * w[2:3, :]
    e_ref[0] = e
    first = pl.program_id(0) == 0
    last = pl.program_id(0) == B - 1

    @pl.when(first)
    def _():
        acc_ref[...] = jnp.zeros_like(acc_ref)

    acc_ref[0:1] = acc_ref[0:1] + jnp.sum(e, axis=0).reshape(1, DIM)
    acc_ref[1:2] = acc_ref[1:2] + jnp.sum(e * e, axis=0).reshape(1, DIM)

    @pl.when(last)
    def _():
        st_ref[...] = acc_ref[...]


def _embed(points, WeT):
    return pl.pallas_call(
        _embed_body,
        grid=(B,),
        in_specs=[
            pl.BlockSpec((1, N, 3), lambda b: (b, 0, 0)),
            pl.BlockSpec((3, DIM), lambda b: (0, 0)),
        ],
        out_specs=[
            pl.BlockSpec((1, N, DIM), lambda b: (b, 0, 0)),
            pl.BlockSpec((8, DIM), lambda b: (0, 0)),
        ],
        out_shape=[
            jax.ShapeDtypeStruct((B, N, DIM), jnp.float32),
            jax.ShapeDtypeStruct((8, DIM), jnp.float32),
        ],
        scratch_shapes=[pltpu.VMEM((8, DIM), jnp.float32)],
    )(points, WeT)


def _bf(x):
    return x.astype(jnp.bfloat16)


def _mm(a, b):
    return lax.dot_general(_bf(a), _bf(b), (((a.ndim - 1,), (0,)), ((), ())),
                           preferred_element_type=jnp.float32)


def _qkv_body(act, xp_ref, sc_ref, sh_ref, wq_ref, wk_ref, wv_ref,
              x_ref, q_ref, k_ref, v_ref):
    x = xp_ref[0] * sc_ref[...] + sh_ref[...]
    if act:
        x = jnp.maximum(x, 0.0)
    x_ref[0] = x
    q_ref[0] = _mm(x, wq_ref[...])
    k_ref[0] = _mm(x, wk_ref[...])
    v_ref[0] = _mm(x, wv_ref[...])


def _qkv(x_pre, sc, sh, act, WqT, WkT, WvT):
    return pl.pallas_call(
        functools.partial(_qkv_body, act),
        grid=(B,),
        in_specs=[
            pl.BlockSpec((1, N, DIM), lambda b: (b, 0, 0)),
            pl.BlockSpec((1, DIM), lambda b: (0, 0)),
            pl.BlockSpec((1, DIM), lambda b: (0, 0)),
            pl.BlockSpec((DIM, DIM), lambda b: (0, 0)),
            pl.BlockSpec((DIM, DIM), lambda b: (0, 0)),
            pl.BlockSpec((DIM, DIM), lambda b: (0, 0)),
        ],
        out_specs=[
            pl.BlockSpec((1, N, DIM), lambda b: (b, 0, 0)),
            pl.BlockSpec((1, N, DIM), lambda b: (b, 0, 0)),
            pl.BlockSpec((1, N, DIM), lambda b: (b, 0, 0)),
            pl.BlockSpec((1, N, DIM), lambda b: (b, 0, 0)),
        ],
        out_shape=[
            jax.ShapeDtypeStruct((B, N, DIM), jnp.float32),
            jax.ShapeDtypeStruct((B, N, DIM), jnp.float32),
            jax.ShapeDtypeStruct((B, N, DIM), jnp.float32),
            jax.ShapeDtypeStruct((B, N, DIM), jnp.float32),
        ],
    )(x_pre, sc, sh, WqT, WkT, WvT)


def _t1_3d(p, xn3, w1):
    t = (p[:, None, 0:1] - xn3[:, :, 0:1]) * w1[0:1, :][None]
    t = t + (p[:, None, 1:2] - xn3[:, :, 1:2]) * w1[1:2, :][None]
    t = t + (p[:, None, 2:3] - xn3[:, :, 2:3]) * w1[2:3, :][None]
    return t


def _dot3(a, w):
    return lax.dot_general(_bf(a), _bf(w), (((2,), (0,)), ((), ())),
                           preferred_element_type=jnp.float32)


def _pe_stats_body(p_ref, xn_ref, w1_ref, st_ref, acc_ref):
    p = p_ref[0]
    xn3 = xn_ref[0].reshape(NB, K, 16)
    t1 = _t1_3d(p, xn3, w1_ref[...])
    s = jnp.sum(t1, axis=(0, 1))
    sq = jnp.sum(t1 * t1, axis=(0, 1))
    _acc_stats(acc_ref, st_ref, s, sq, NBLK - 1)


def _pe_stats(points, xyznn, Wpe1T):
    return pl.pallas_call(
        _pe_stats_body,
        grid=(B, NBLK),
        in_specs=[
            pl.BlockSpec((1, NB, 3), lambda b, r: (b, r, 0)),
            pl.BlockSpec((1, NBK, 16), lambda b, r: (b, r, 0)),
            pl.BlockSpec((3, DIM), lambda b, r: (0, 0)),
        ],
        out_specs=pl.BlockSpec((8, DIM), lambda b, r: (0, 0)),
        out_shape=jax.ShapeDtypeStruct((8, DIM), jnp.float32),
        scratch_shapes=[pltpu.VMEM((8, DIM), jnp.float32)],
    )(points, xyznn, Wpe1T)


def _pe_a1_t2(p_ref, xn_ref, q_ref, k_ref, w1_ref, sc1_ref, sh1_ref,
              w2_ref, wa1_ref):
    p = p_ref[0]
    xn3 = xn_ref[0].reshape(NB, K, 16)
    t1 = _t1_3d(p, xn3, w1_ref[...])
    pe_in = jnp.maximum(t1 * sc1_ref[...][None] + sh1_ref[...][None], 0.0)
    pe = _dot3(pe_in, w2_ref[...])
    q3 = q_ref[0][:, None, :]
    k3 = k_ref[0].reshape(NB, K, DIM)
    a1 = (q3 - k3) + pe
    t2 = _dot3(a1, wa1_ref[...])
    return pe, t2


def _t2_stats_body(p_ref, xn_ref, q_ref, k_ref, w1_ref, sc1_ref, sh1_ref,
                   w2_ref, wa1_ref, st_ref, acc_ref):
    _, t2 = _pe_a1_t2(p_ref, xn_ref, q_ref, k_ref, w1_ref, sc1_ref, sh1_ref,
                      w2_ref, wa1_ref)
    s = jnp.sum(t2, axis=(0, 1))
    sq = jnp.sum(t2 * t2, axis=(0, 1))
    _acc_stats(acc_ref, st_ref, s, sq, NBLK - 1)


def _t2_stats(points, xyznn, q, knn_, Wpe1T, sc1, sh1, Wpe2T, Wam1T):
    return pl.pallas_call(
        _t2_stats_body,
        grid=(B, NBLK),
        in_specs=[
            pl.BlockSpec((1, NB, 3), lambda b, r: (b, r, 0)),
            pl.BlockSpec((1, NBK, 16), lambda b, r: (b, r, 0)),
            pl.BlockSpec((1, NB, DIM), lambda b, r: (b, r, 0)),
            pl.BlockSpec((1, NBK, DIM), lambda b, r: (b, r, 0)),
            pl.BlockSpec((3, DIM), lambda b, r: (0, 0)),
            pl.BlockSpec((1, DIM), lambda b, r: (0, 0)),
            pl.BlockSpec((1, DIM), lambda b, r: (0, 0)),
            pl.BlockSpec((DIM, DIM), lambda b, r: (0, 0)),
            pl.BlockSpec((DIM, DIM), lambda b, r: (0, 0)),
        ],
        out_specs=pl.BlockSpec((8, DIM), lambda b, r: (0, 0)),
        out_shape=jax.ShapeDtypeStruct((8, DIM), jnp.float32),
        scratch_shapes=[pltpu.VMEM((8, DIM), jnp.float32)],
    )(points, xyznn, q, knn_, Wpe1T, sc1, sh1, Wpe2T, Wam1T)


def _attn_out_body(p_ref, xn_ref, q_ref, k_ref, v_ref, x_ref, w1_ref,
                   sc1_ref, sh1_ref, w2_ref, wa1_ref, sc2_ref, sh2_ref,
                   wa2_ref, r_ref, st_ref, acc_ref):
    pe, t2 = _pe_a1_t2(p_ref, xn_ref, q_ref, k_ref, w1_ref, sc1_ref, sh1_ref,
                       w2_ref, wa1_ref)
    am_in = jnp.maximum(t2 * sc2_ref[...][None] + sh2_ref[...][None], 0.0)
    al = _dot3(am_in, wa2_ref[...])
    m = jnp.max(al, axis=1, keepdims=True)
    e = jnp.exp(al - m)
    attn = e / jnp.sum(e, axis=1, keepdims=True)
    v3 = v_ref[0].reshape(NB, K, DIM)
    out = jnp.sum(attn * (v3 + pe), axis=1)
    r = x_ref[0] + out
    r_ref[0] = r
    s = jnp.sum(r, axis=0)
    sq = jnp.sum(r * r, axis=0)
    _acc_stats(acc_ref, st_ref, s, sq, NBLK - 1)


def _attn_out(points, xyznn, q, knn_, vnn_, x, Wpe1T, sc1, sh1, Wpe2T, Wam1T,
              sc2, sh2, Wam2T):
    return pl.pallas_call(
        _attn_out_body,
        grid=(B, NBLK),
        in_specs=[
            pl.BlockSpec((1, NB, 3), lambda b, r: (b, r, 0)),
            pl.BlockSpec((1, NBK, 16), lambda b, r: (b, r, 0)),
            pl.BlockSpec((1, NB, DIM), lambda b, r: (b, r, 0)),
            pl.BlockSpec((1, NBK, DIM), lambda b, r: (b, r, 0)),
            pl.BlockSpec((1, NBK, DIM), lambda b, r: (b, r, 0)),
            pl.BlockSpec((1, NB, DIM), lambda b, r: (b, r, 0)),
            pl.BlockSpec((3, DIM), lambda b, r: (0, 0)),
            pl.BlockSpec((1, DIM), lambda b, r: (0, 0)),
            pl.BlockSpec((1, DIM), lambda b, r: (0, 0)),
            pl.BlockSpec((DIM, DIM), lambda b, r: (0, 0)),
            pl.BlockSpec((DIM, DIM), lambda b, r: (0, 0)),
            pl.BlockSpec((1, DIM), lambda b, r: (0, 0)),
            pl.BlockSpec((1, DIM), lambda b, r: (0, 0)),
            pl.BlockSpec((DIM, DIM), lambda b, r: (0, 0)),
        ],
        out_specs=[
            pl.BlockSpec((1, NB, DIM), lambda b, r: (b, r, 0)),
            pl.BlockSpec((8, DIM), lambda b, r: (0, 0)),
        ],
        out_shape=[
            jax.ShapeDtypeStruct((B, N, DIM), jnp.float32),
            jax.ShapeDtypeStruct((8, DIM), jnp.float32),
        ],
        scratch_shapes=[pltpu.VMEM((8, DIM), jnp.float32)],
    )(points, xyznn, q, knn_, vnn_, x, Wpe1T, sc1, sh1, Wpe2T, Wam1T,
      sc2, sh2, Wam2T)


def _ffn1_body(r_ref, sc_ref, sh_ref, wf1_ref, st_ref, acc_ref):
    x1 = r_ref[0] * sc_ref[...] + sh_ref[...]
    t3 = _mm(x1, wf1_ref[...])
    first = pl.program_id(0) == 0
    last = pl.program_id(0) == B - 1

    @pl.when(first)
    def _():
        acc_ref[...] = jnp.zeros_like(acc_ref)

    acc_ref[0:1] = acc_ref[0:1] + jnp.sum(t3, axis=0).reshape(1, HID)
    acc_ref[1:2] = acc_ref[1:2] + jnp.sum(t3 * t3, axis=0).reshape(1, HID)

    @pl.when(last)
    def _():
        st_ref[...] = acc_ref[...]


def _ffn1(r, sc3, sh3, Wf1T):
    return pl.pallas_call(
        _ffn1_body,
        grid=(B,),
        in_specs=[
            pl.BlockSpec((1, N, DIM), lambda b: (b, 0, 0)),
            pl.BlockSpec((1, DIM), lambda b: (0, 0)),
            pl.BlockSpec((1, DIM), lambda b: (0, 0)),
            pl.BlockSpec((DIM, HID), lambda b: (0, 0)),
        ],
        out_specs=pl.BlockSpec((8, HID), lambda b: (0, 0)),
        out_shape=jax.ShapeDtypeStruct((8, HID), jnp.float32),
        scratch_shapes=[pltpu.VMEM((8, HID), jnp.float32)],
    )(r, sc3, sh3, Wf1T)


def _ffn2_body(r_ref, sc3_ref, sh3_ref, wf1_ref, sc4_ref, sh4_ref, wf2_ref,
               r2_ref, st_ref, acc_ref):
    x1 = r_ref[0] * sc3_ref[...] + sh3_ref[...]
    t3 = _mm(x1, wf1_ref[...])
    h = jnp.maximum(t3 * sc4_ref[...] + sh4_ref[...], 0.0)
    t4 = _mm(h, wf2_ref[...])
    r2 = x1 + t4
    r2_ref[0] = r2
    first = pl.program_id(0) == 0
    last = pl.program_id(0) == B - 1

    @pl.when(first)
    def _():
        acc_ref[...] = jnp.zeros_like(acc_ref)

    acc_ref[0:1] = acc_ref[0:1] + jnp.sum(r2, axis=0).reshape(1, DIM)
    acc_ref[1:2] = acc_ref[1:2] + jnp.sum(r2 * r2, axis=0).reshape(1, DIM)

    @pl.when(last)
    def _():
        st_ref[...] = acc_ref[...]


def _ffn2(r, sc3, sh3, Wf1T, sc4, sh4, Wf2T):
    return pl.pallas_call(
        _ffn2_body,
        grid=(B,),
        in_specs=[
            pl.BlockSpec((1, N, DIM), lambda b: (b, 0, 0)),
            pl.BlockSpec((1, DIM), lambda b: (0, 0)),
            pl.BlockSpec((1, DIM), lambda b: (0, 0)),
            pl.BlockSpec((DIM, HID), lambda b: (0, 0)),
            pl.BlockSpec((1, HID), lambda b: (0, 0)),
            pl.BlockSpec((1, HID), lambda b: (0, 0)),
            pl.BlockSpec((HID, DIM), lambda b: (0, 0)),
        ],
        out_specs=[
            pl.BlockSpec((1, N, DIM), lambda b: (b, 0, 0)),
            pl.BlockSpec((8, DIM), lambda b: (0, 0)),
        ],
        out_shape=[
            jax.ShapeDtypeStruct((B, N, DIM), jnp.float32),
            jax.ShapeDtypeStruct((8, DIM), jnp.float32),
        ],
        scratch_shapes=[pltpu.VMEM((8, DIM), jnp.float32)],
    )(r, sc3, sh3, Wf1T, sc4, sh4, Wf2T)


def _head_body(r2_ref, sc_ref, sh_ref, ws1_ref, gs_ref, bs_ref, ws2_ref,
               bs2_ref, o_ref):
    ws1 = ws1_ref[...]
    hs = []
    s = jnp.zeros((1, HID), jnp.float32)
    sq = jnp.zeros((1, HID), jnp.float32)
    for b in range(B):
        x = r2_ref[b] * sc_ref[...] + sh_ref[...]
        h = _mm(x, ws1)
        hs.append(h)
        s = s + jnp.sum(h, axis=0).reshape(1, HID)
        sq = sq + jnp.sum(h * h, axis=0).reshape(1, HID)
    cnt = float(B * N)
    m = s / cnt
    v = sq / cnt - m * m
    scale = gs_ref[...] / jnp.sqrt(v + EPS)
    shift = bs_ref[...] - m * scale
    ws2 = ws2_ref[...]
    for b in range(B):
        h = jnp.maximum(hs[b] * scale + shift, 0.0)
        o = lax.dot_general(_bf(ws2), _bf(h), (((1,), (1,)), ((), ())),
                            preferred_element_type=jnp.float32)
        o_ref[b] = o + bs2_ref[...]


def _head(r2, sc5, sh5, Ws1T, g_s, b_s, Ws2, bs2c):
    return pl.pallas_call(
        _head_body,
        out_shape=jax.ShapeDtypeStruct((B, NC, N), jnp.float32),
    )(r2, sc5, sh5, Ws1T, g_s, b_s, Ws2, bs2c)


def _aff(st, cnt, g, b):
    s, sq = st[0, :g.shape[0]], st[1, :g.shape[0]]
    m = s / cnt
    v = sq / cnt - m * m
    sc = g / jnp.sqrt(v + EPS)
    sh = b - m * sc
    return sc.reshape(1, -1), sh.reshape(1, -1)


def kernel(points, We, g_e, b_e, Wq, Wk, Wv, Wpe1, g_pe, b_pe, Wpe2, Wam1,
           g_am, b_am, Wam2, g1, b1, Wf1, g_f, b_f, Wf2, g2, b2, Ws1, g_s,
           b_s, Ws2, bs2):
    pT = jnp.transpose(points, (0, 2, 1))
    idx = _knn(points, pT)
    idxf = idx.reshape(B * N * K)

    xyztab = jnp.pad(points.reshape(B * N, 3), ((0, 0), (0, 13)))
    xyznn = _make_sc_gather(B * N, 16, B * N * K)(xyztab, idxf)
    xyznn = xyznn.reshape(B, N * K, 16)

    e_pre, st_e = _embed(points, We.T)
    sc, sh = _aff(st_e, B * N, g_e, b_e)
    x_pre, act = e_pre, True

    f_gather = _make_sc_gather(B * N, DIM, B * N * K)
    for i in range(DEPTH):
        x, q, k, v = _qkv(x_pre, sc, sh, act, Wq[i].T, Wk[i].T, Wv[i].T)
        knn_ = f_gather(k.reshape(B * N, DIM), idxf).reshape(B, N * K, DIM)
        vnn_ = f_gather(v.reshape(B * N, DIM), idxf).reshape(B, N * K, DIM)
        w1T, w2T, wa1T, wa2T = Wpe1[i].T, Wpe2[i].T, Wam1[i].T, Wam2[i].T
        st1 = _pe_stats(points, xyznn, w1T)
        sc1, sh1 = _aff(st1, B * N * K, g_pe[i], b_pe[i])
        st2 = _t2_stats(points, xyznn, q, knn_, w1T, sc1, sh1, w2T, wa1T)
        sc2, sh2 = _aff(st2, B * N * K, g_am[i], b_am[i])
        r, st3 = _attn_out(points, xyznn, q, knn_, vnn_, x, w1T, sc1, sh1,
                           w2T, wa1T, sc2, sh2, wa2T)
        sc3, sh3 = _aff(st3, B * N, g1[i], b1[i])
        st4 = _ffn1(r, sc3, sh3, Wf1[i].T)
        sc4, sh4 = _aff(st4, B * N, g_f[i], b_f[i])
        r2, st5 = _ffn2(r, sc3, sh3, Wf1[i].T, sc4, sh4, Wf2[i].T)
        sc, sh = _aff(st5, B * N, g2[i], b2[i])
        x_pre, act = r2, False

    return _head(x_pre, sc, sh, Ws1.T, g_s.reshape(1, HID),
                 b_s.reshape(1, HID), Ws2, bs2.reshape(NC, 1))


# speedup vs baseline: 11.8176x; 1.1578x over previous
import functools

import jax
import jax.numpy as jnp
from jax import lax
from jax.experimental import pallas as pl
from jax.experimental.pallas import tpu as pltpu
from jax.experimental.pallas import tpu_sc as plsc

B, N, DIM, DEPTH, K, HID, NC = 4, 2048, 64, 2, 16, 128, 13
EPS = 1e-5
RB = 256
NB = 256
NBLK = N // NB
NBK = NB * K


def _knn_body(p_ref, pt_ref, o_ref):
    p = p_ref[0]
    pt = pt_ref[0]
    sqr = jnp.sum(p * p, axis=1, keepdims=True)
    sqa = jnp.sum(pt * pt, axis=0, keepdims=True)
    dot = lax.dot_general(p.astype(jnp.bfloat16), pt.astype(jnp.bfloat16),
                          (((1,), (0,)), ((), ())),
                          preferred_element_type=jnp.float32)
    d = (sqr + sqa) - 2.0 * dot
    cols = lax.broadcasted_iota(jnp.int32, (RB, N), 1)
    kcols = lax.broadcasted_iota(jnp.int32, (RB, K), 1)
    kidx = jnp.zeros((RB, K), jnp.int32)
    for j in range(K):
        m = jnp.min(d, axis=1, keepdims=True)
        am = jnp.min(jnp.where(d == m, cols, N), axis=1, keepdims=True)
        kidx = jnp.where(kcols == j, am, kidx)
        d = jnp.where(cols == am, jnp.float32(jnp.inf), d)
    o_ref[0] = kidx + pl.program_id(0) * N


def _knn(points, pT):
    return pl.pallas_call(
        _knn_body,
        grid=(B, N // RB),
        in_specs=[
            pl.BlockSpec((1, RB, 3), lambda b, r: (b, r, 0)),
            pl.BlockSpec((1, 3, N), lambda b, r: (b, 0, 0)),
        ],
        out_specs=pl.BlockSpec((1, RB, K), lambda b, r: (b, r, 0)),
        out_shape=jax.ShapeDtypeStruct((B, N, K), jnp.int32),
    )(points, pT)


def _make_sc_gather(R, D, M):
    NW = 32
    Mw = M // NW
    CH = min(Mw, 65536 // D)
    NCH = Mw // CH
    mesh = plsc.VectorSubcoreMesh(core_axis_name="c", subcore_axis_name="s")

    @functools.partial(
        pl.kernel,
        mesh=mesh,
        compiler_params=pltpu.CompilerParams(use_tc_tiling_on_sc=False),
        out_type=jax.ShapeDtypeStruct((M, D), jnp.float32),
        scratch_types=[
            pltpu.VMEM((CH,), jnp.int32),
            pltpu.VMEM((CH, D), jnp.float32),
            pltpu.SemaphoreType.DMA,
        ],
    )
    def g(tab_hbm, idx_hbm, out_hbm, idx_v, rows_v, sem):
        wid = lax.axis_index("s") * 2 + lax.axis_index("c")
        base = wid * Mw
        for c in range(NCH):
            off = base + c * CH
            pltpu.sync_copy(idx_hbm.at[pl.ds(off, CH)], idx_v)
            pltpu.async_copy(tab_hbm.at[idx_v], rows_v, sem).wait()
            pltpu.sync_copy(rows_v, out_hbm.at[pl.ds(off, CH)])

    return g


def _acc_stats(acc_ref, st_ref, s, sq, nblk_last):
    first = (pl.program_id(0) == 0) & (pl.program_id(1) == 0)
    last = (pl.program_id(0) == B - 1) & (pl.program_id(1) == nblk_last)

    @pl.when(first)
    def _():
        acc_ref[...] = jnp.zeros_like(acc_ref)

    c = s.shape[-1]
    acc_ref[0:1] = acc_ref[0:1] + s.reshape(1, c)
    acc_ref[1:2] = acc_ref[1:2] + sq.reshape(1, c)

    @pl.when(last)
    def _():
        st_ref[...] = acc_ref[...]


def _embed_body(p_ref, w_ref, e_ref, st_ref, acc_ref):
    p = p_ref[0]
    w = w_ref[...]
    e = lax.dot_general(p.astype(jnp.bfloat16), w.astype(jnp.bfloat16),
                        (((1,), (0,)), ((), ())),
                        preferred_element_type=jnp.float32)
    e_ref[0] = e
    first = pl.program_id(0) == 0
    last = pl.program_id(0) == B - 1

    @pl.when(first)
    def _():
        acc_ref[...] = jnp.zeros_like(acc_ref)

    acc_ref[0:1] = acc_ref[0:1] + jnp.sum(e, axis=0).reshape(1, DIM)
    acc_ref[1:2] = acc_ref[1:2] + jnp.sum(e * e, axis=0).reshape(1, DIM)

    @pl.when(last)
    def _():
        st_ref[...] = acc_ref[...]


def _embed(points, WeT):
    return pl.pallas_call(
        _embed_body,
        grid=(B,),
        in_specs=[
            pl.BlockSpec((1, N, 3), lambda b: (b, 0, 0)),
            pl.BlockSpec((3, DIM), lambda b: (0, 0)),
        ],
        out_specs=[
            pl.BlockSpec((1, N, DIM), lambda b: (b, 0, 0)),
            pl.BlockSpec((8, DIM), lambda b: (0, 0)),
        ],
        out_shape=[
            jax.ShapeDtypeStruct((B, N, DIM), jnp.float32),
            jax.ShapeDtypeStruct((8, DIM), jnp.float32),
        ],
        scratch_shapes=[pltpu.VMEM((8, DIM), jnp.float32)],
    )(points, WeT)


def _bf(x):
    return x.astype(jnp.bfloat16)


def _mm(a, b):
    return lax.dot_general(_bf(a), _bf(b), (((a.ndim - 1,), (0,)), ((), ())),
                           preferred_element_type=jnp.float32)


def _qkv_body(act, xp_ref, sc_ref, sh_ref, wq_ref, wk_ref, wv_ref,
              x_ref, q_ref, k_ref, v_ref):
    x = xp_ref[0] * sc_ref[...] + sh_ref[...]
    if act:
        x = jnp.maximum(x, 0.0)
    x_ref[0] = x
    q_ref[0] = _mm(x, wq_ref[...])
    k_ref[0] = _mm(x, wk_ref[...])
    v_ref[0] = _mm(x, wv_ref[...])


def _qkv(x_pre, sc, sh, act, WqT, WkT, WvT):
    return pl.pallas_call(
        functools.partial(_qkv_body, act),
        grid=(B,),
        in_specs=[
            pl.BlockSpec((1, N, DIM), lambda b: (b, 0, 0)),
            pl.BlockSpec((1, DIM), lambda b: (0, 0)),
            pl.BlockSpec((1, DIM), lambda b: (0, 0)),
            pl.BlockSpec((DIM, DIM), lambda b: (0, 0)),
            pl.BlockSpec((DIM, DIM), lambda b: (0, 0)),
            pl.BlockSpec((DIM, DIM), lambda b: (0, 0)),
        ],
        out_specs=[
            pl.BlockSpec((1, N, DIM), lambda b: (b, 0, 0)),
            pl.BlockSpec((1, N, DIM), lambda b: (b, 0, 0)),
            pl.BlockSpec((1, N, DIM), lambda b: (b, 0, 0)),
            pl.BlockSpec((1, N, DIM), lambda b: (b, 0, 0)),
        ],
        out_shape=[
            jax.ShapeDtypeStruct((B, N, DIM), jnp.float32),
            jax.ShapeDtypeStruct((B, N, DIM), jnp.float32),
            jax.ShapeDtypeStruct((B, N, DIM), jnp.float32),
            jax.ShapeDtypeStruct((B, N, DIM), jnp.float32),
        ],
    )(x_pre, sc, sh, WqT, WkT, WvT)


def _dot3(a, w):
    return lax.dot_general(_bf(a), _bf(w), (((2,), (0,)), ((), ())),
                           preferred_element_type=jnp.float32)


def _pd_body(p_ref, xn_ref, pd_ref, st_ref, acc_ref):
    p16 = p_ref[0]
    xn = xn_ref[0]
    pd = (p16[:, None, :] - xn.reshape(NB, K, 16)).reshape(NBK, 16)
    pd_ref[0] = pd
    s1 = jnp.sum(pd, axis=0)
    s2 = jnp.sum(pd * pd, axis=0)
    r1 = jnp.sum(pd * jnp.roll(pd, -1, axis=1), axis=0)
    r2 = jnp.sum(pd * jnp.roll(pd, -2, axis=1), axis=0)
    first = (pl.program_id(0) == 0) & (pl.program_id(1) == 0)
    last = (pl.program_id(0) == B - 1) & (pl.program_id(1) == NBLK - 1)

    @pl.when(first)
    def _():
        acc_ref[...] = jnp.zeros_like(acc_ref)

    acc_ref[0:1] = acc_ref[0:1] + s1.reshape(1, 16)
    acc_ref[1:2] = acc_ref[1:2] + s2.reshape(1, 16)
    acc_ref[2:3] = acc_ref[2:3] + r1.reshape(1, 16)
    acc_ref[3:4] = acc_ref[3:4] + r2.reshape(1, 16)

    @pl.when(last)
    def _():
        st_ref[...] = acc_ref[...]


def _pos_diff(points16, xyznn):
    return pl.pallas_call(
        _pd_body,
        grid=(B, NBLK),
        in_specs=[
            pl.BlockSpec((1, NB, 16), lambda b, r: (b, r, 0)),
            pl.BlockSpec((1, NBK, 16), lambda b, r: (b, r, 0)),
        ],
        out_specs=[
            pl.BlockSpec((1, NBK, 16), lambda b, r: (b, r, 0)),
            pl.BlockSpec((8, 16), lambda b, r: (0, 0)),
        ],
        out_shape=[
            jax.ShapeDtypeStruct((B, N * K, 16), jnp.float32),
            jax.ShapeDtypeStruct((8, 16), jnp.float32),
        ],
        scratch_shapes=[pltpu.VMEM((8, 16), jnp.float32)],
    )(points16, xyznn)


def _pe_a1_t2(pd_ref, q_ref, k_ref, w1_ref, sc1_ref, sh1_ref,
              w2_ref, wa1_ref):
    t1 = _mm(pd_ref[0], w1_ref[...])
    pe_in = jnp.maximum(t1 * sc1_ref[...] + sh1_ref[...], 0.0)
    pe = _mm(pe_in, w2_ref[...])
    q3 = q_ref[0][:, None, :]
    k3 = k_ref[0].reshape(NB, K, DIM)
    a1 = (q3 - k3) + pe.reshape(NB, K, DIM)
    t2 = _dot3(a1, wa1_ref[...])
    return pe, t2


def _t2_stats_body(pd_ref, q_ref, k_ref, w1_ref, sc1_ref, sh1_ref,
                   w2_ref, wa1_ref, st_ref, acc_ref):
    _, t2 = _pe_a1_t2(pd_ref, q_ref, k_ref, w1_ref, sc1_ref, sh1_ref,
                      w2_ref, wa1_ref)
    s = jnp.sum(t2, axis=(0, 1))
    sq = jnp.sum(t2 * t2, axis=(0, 1))
    _acc_stats(acc_ref, st_ref, s, sq, NBLK - 1)


def _t2_stats(pd, q, knn_, Wpe1T16, sc1, sh1, Wpe2T, Wam1T):
    return pl.pallas_call(
        _t2_stats_body,
        grid=(B, NBLK),
        in_specs=[
            pl.BlockSpec((1, NBK, 16), lambda b, r: (b, r, 0)),
            pl.BlockSpec((1, NB, DIM), lambda b, r: (b, r, 0)),
            pl.BlockSpec((1, NBK, DIM), lambda b, r: (b, r, 0)),
            pl.BlockSpec((16, DIM), lambda b, r: (0, 0)),
            pl.BlockSpec((1, DIM), lambda b, r: (0, 0)),
            pl.BlockSpec((1, DIM), lambda b, r: (0, 0)),
            pl.BlockSpec((DIM, DIM), lambda b, r: (0, 0)),
            pl.BlockSpec((DIM, DIM), lambda b, r: (0, 0)),
        ],
        out_specs=pl.BlockSpec((8, DIM), lambda b, r: (0, 0)),
        out_shape=jax.ShapeDtypeStruct((8, DIM), jnp.float32),
        scratch_shapes=[pltpu.VMEM((8, DIM), jnp.float32)],
    )(pd, q, knn_, Wpe1T16, sc1, sh1, Wpe2T, Wam1T)


def _attn_out_body(pd_ref, q_ref, k_ref, v_ref, x_ref, w1_ref,
                   sc1_ref, sh1_ref, w2_ref, wa1_ref, sc2_ref, sh2_ref,
                   wa2_ref, r_ref, st_ref, acc_ref):
    pe, t2 = _pe_a1_t2(pd_ref, q_ref, k_ref, w1_ref, sc1_ref, sh1_ref,
                       w2_ref, wa1_ref)
    am_in = jnp.maximum(t2 * sc2_ref[...][None] + sh2_ref[...][None], 0.0)
    al = _dot3(am_in, wa2_ref[...])
    m = jnp.max(al, axis=1, keepdims=True)
    e = jnp.exp(al - m)
    attn = e / jnp.sum(e, axis=1, keepdims=True)
    v3 = v_ref[0].reshape(NB, K, DIM)
    out = jnp.sum(attn * (v3 + pe.reshape(NB, K, DIM)), axis=1)
    r = x_ref[0] + out
    r_ref[0] = r
    s = jnp.sum(r, axis=0)
    sq = jnp.sum(r * r, axis=0)
    _acc_stats(acc_ref, st_ref, s, sq, NBLK - 1)


def _attn_out(pd, q, knn_, vnn_, x, Wpe1T16, sc1, sh1, Wpe2T, Wam1T,
              sc2, sh2, Wam2T):
    return pl.pallas_call(
        _attn_out_body,
        grid=(B, NBLK),
        in_specs=[
            pl.BlockSpec((1, NBK, 16), lambda b, r: (b, r, 0)),
            pl.BlockSpec((1, NB, DIM), lambda b, r: (b, r, 0)),
            pl.BlockSpec((1, NBK, DIM), lambda b, r: (b, r, 0)),
            pl.BlockSpec((1, NBK, DIM), lambda b, r: (b, r, 0)),
            pl.BlockSpec((1, NB, DIM), lambda b, r: (b, r, 0)),
            pl.BlockSpec((16, DIM), lambda b, r: (0, 0)),
            pl.BlockSpec((1, DIM), lambda b, r: (0, 0)),
            pl.BlockSpec((1, DIM), lambda b, r: (0, 0)),
            pl.BlockSpec((DIM, DIM), lambda b, r: (0, 0)),
            pl.BlockSpec((DIM, DIM), lambda b, r: (0, 0)),
            pl.BlockSpec((1, DIM), lambda b, r: (0, 0)),
            pl.BlockSpec((1, DIM), lambda b, r: (0, 0)),
            pl.BlockSpec((DIM, DIM), lambda b, r: (0, 0)),
        ],
        out_specs=[
            pl.BlockSpec((1, NB, DIM), lambda b, r: (b, r, 0)),
            pl.BlockSpec((8, DIM), lambda b, r: (0, 0)),
        ],
        out_shape=[
            jax.ShapeDtypeStruct((B, N, DIM), jnp.float32),
            jax.ShapeDtypeStruct((8, DIM), jnp.float32),
        ],
        scratch_shapes=[pltpu.VMEM((8, DIM), jnp.float32)],
    )(pd, q, knn_, vnn_, x, Wpe1T16, sc1, sh1, Wpe2T, Wam1T,
      sc2, sh2, Wam2T)


def _ffn1_body(r_ref, sc_ref, sh_ref, wf1_ref, st_ref, acc_ref):
    x1 = r_ref[0] * sc_ref[...] + sh_ref[...]
    t3 = _mm(x1, wf1_ref[...])
    first = pl.program_id(0) == 0
    last = pl.program_id(0) == B - 1

    @pl.when(first)
    def _():
        acc_ref[...] = jnp.zeros_like(acc_ref)

    acc_ref[0:1] = acc_ref[0:1] + jnp.sum(t3, axis=0).reshape(1, HID)
    acc_ref[1:2] = acc_ref[1:2] + jnp.sum(t3 * t3, axis=0).reshape(1, HID)

    @pl.when(last)
    def _():
        st_ref[...] = acc_ref[...]


def _ffn1(r, sc3, sh3, Wf1T):
    return pl.pallas_call(
        _ffn1_body,
        grid=(B,),
        in_specs=[
            pl.BlockSpec((1, N, DIM), lambda b: (b, 0, 0)),
            pl.BlockSpec((1, DIM), lambda b: (0, 0)),
            pl.BlockSpec((1, DIM), lambda b: (0, 0)),
            pl.BlockSpec((DIM, HID), lambda b: (0, 0)),
        ],
        out_specs=pl.BlockSpec((8, HID), lambda b: (0, 0)),
        out_shape=jax.ShapeDtypeStruct((8, HID), jnp.float32),
        scratch_shapes=[pltpu.VMEM((8, HID), jnp.float32)],
    )(r, sc3, sh3, Wf1T)


def _ffn2_body(r_ref, sc3_ref, sh3_ref, wf1_ref, sc4_ref, sh4_ref, wf2_ref,
               r2_ref, st_ref, acc_ref):
    x1 = r_ref[0] * sc3_ref[...] + sh3_ref[...]
    t3 = _mm(x1, wf1_ref[...])
    h = jnp.maximum(t3 * sc4_ref[...] + sh4_ref[...], 0.0)
    t4 = _mm(h, wf2_ref[...])
    r2 = x1 + t4
    r2_ref[0] = r2
    first = pl.program_id(0) == 0
    last = pl.program_id(0) == B - 1

    @pl.when(first)
    def _():
        acc_ref[...] = jnp.zeros_like(acc_ref)

    acc_ref[0:1] = acc_ref[0:1] + jnp.sum(r2, axis=0).reshape(1, DIM)
    acc_ref[1:2] = acc_ref[1:2] + jnp.sum(r2 * r2, axis=0).reshape(1, DIM)

    @pl.when(last)
    def _():
        st_ref[...] = acc_ref[...]


def _ffn2(r, sc3, sh3, Wf1T, sc4, sh4, Wf2T):
    return pl.pallas_call(
        _ffn2_body,
        grid=(B,),
        in_specs=[
            pl.BlockSpec((1, N, DIM), lambda b: (b, 0, 0)),
            pl.BlockSpec((1, DIM), lambda b: (0, 0)),
            pl.BlockSpec((1, DIM), lambda b: (0, 0)),
            pl.BlockSpec((DIM, HID), lambda b: (0, 0)),
            pl.BlockSpec((1, HID), lambda b: (0, 0)),
            pl.BlockSpec((1, HID), lambda b: (0, 0)),
            pl.BlockSpec((HID, DIM), lambda b: (0, 0)),
        ],
        out_specs=[
            pl.BlockSpec((1, N, DIM), lambda b: (b, 0, 0)),
            pl.BlockSpec((8, DIM), lambda b: (0, 0)),
        ],
        out_shape=[
            jax.ShapeDtypeStruct((B, N, DIM), jnp.float32),
            jax.ShapeDtypeStruct((8, DIM), jnp.float32),
        ],
        scratch_shapes=[pltpu.VMEM((8, DIM), jnp.float32)],
    )(r, sc3, sh3, Wf1T, sc4, sh4, Wf2T)


def _head_body(r2_ref, sc_ref, sh_ref, ws1_ref, gs_ref, bs_ref, ws2_ref,
               bs2_ref, o_ref):
    ws1 = ws1_ref[...]
    hs = []
    s = jnp.zeros((1, HID), jnp.float32)
    sq = jnp.zeros((1, HID), jnp.float32)
    for b in range(B):
        x = r2_ref[b] * sc_ref[...] + sh_ref[...]
        h = _mm(x, ws1)
        hs.append(h)
        s = s + jnp.sum(h, axis=0).reshape(1, HID)
        sq = sq + jnp.sum(h * h, axis=0).reshape(1, HID)
    cnt = float(B * N)
    m = s / cnt
    v = sq / cnt - m * m
    scale = gs_ref[...] / jnp.sqrt(v + EPS)
    shift = bs_ref[...] - m * scale
    ws2 = ws2_ref[...]
    for b in range(B):
        h = jnp.maximum(hs[b] * scale + shift, 0.0)
        o = lax.dot_general(_bf(ws2), _bf(h), (((1,), (1,)), ((), ())),
                            preferred_element_type=jnp.float32)
        o_ref[b] = o + bs2_ref[...]


def _head(r2, sc5, sh5, Ws1T, g_s, b_s, Ws2, bs2c):
    return pl.pallas_call(
        _head_body,
        out_shape=jax.ShapeDtypeStruct((B, NC, N), jnp.float32),
    )(r2, sc5, sh5, Ws1T, g_s, b_s, Ws2, bs2c)


def _aff(st, cnt, g, b):
    s, sq = st[0, :g.shape[0]], st[1, :g.shape[0]]
    m = s / cnt
    v = sq / cnt - m * m
    sc = g / jnp.sqrt(v + EPS)
    sh = b - m * sc
    return sc.reshape(1, -1), sh.reshape(1, -1)


def kernel(points, We, g_e, b_e, Wq, Wk, Wv, Wpe1, g_pe, b_pe, Wpe2, Wam1,
           g_am, b_am, Wam2, g1, b1, Wf1, g_f, b_f, Wf2, g2, b2, Ws1, g_s,
           b_s, Ws2, bs2):
    pT = jnp.transpose(points, (0, 2, 1))
    idx = _knn(points, pT)
    idxf = idx.reshape(B * N * K)

    xyztab = jnp.pad(points.reshape(B * N, 3), ((0, 0), (0, 13)))
    xyznn = _make_sc_gather(B * N, 16, B * N * K)(xyztab, idxf)
    xyznn = xyznn.reshape(B, N * K, 16)

    pd, st_pd = _pos_diff(xyztab.reshape(B, N, 16), xyznn)
    cnt_pd = float(B * N * K)
    mu3 = st_pd[0, :3] / cnt_pd
    d0, d1, d2_ = st_pd[1, 0], st_pd[1, 1], st_pd[1, 2]
    xy, yz, xz = st_pd[2, 0], st_pd[2, 1], st_pd[3, 0]
    Smat = jnp.stack([
        jnp.stack([d0, xy, xz]),
        jnp.stack([xy, d1, yz]),
        jnp.stack([xz, yz, d2_]),
    ])

    e_pre, st_e = _embed(points, We.T)
    sc, sh = _aff(st_e, B * N, g_e, b_e)
    x_pre, act = e_pre, True

    f_gather = _make_sc_gather(B * N, DIM, B * N * K)
    for i in range(DEPTH):
        x, q, k, v = _qkv(x_pre, sc, sh, act, Wq[i].T, Wk[i].T, Wv[i].T)
        knn_ = f_gather(k.reshape(B * N, DIM), idxf).reshape(B, N * K, DIM)
        vnn_ = f_gather(v.reshape(B * N, DIM), idxf).reshape(B, N * K, DIM)
        w1T, w2T, wa1T, wa2T = Wpe1[i].T, Wpe2[i].T, Wam1[i].T, Wam2[i].T
        w1T16 = jnp.pad(w1T, ((0, 13), (0, 0)))
        m1 = mu3 @ w1T
        e2 = jnp.sum(w1T * (Smat @ w1T), axis=0) / cnt_pd
        v1 = e2 - m1 * m1
        sc1 = (g_pe[i] / jnp.sqrt(v1 + EPS)).reshape(1, DIM)
        sh1 = (b_pe[i] - m1 * sc1[0]).reshape(1, DIM)
        st2 = _t2_stats(pd, q, knn_, w1T16, sc1, sh1, w2T, wa1T)
        sc2, sh2 = _aff(st2, B * N * K, g_am[i], b_am[i])
        r, st3 = _attn_out(pd, q, knn_, vnn_, x, w1T16, sc1, sh1,
                           w2T, wa1T, sc2, sh2, wa2T)
        sc3, sh3 = _aff(st3, B * N, g1[i], b1[i])
        st4 = _ffn1(r, sc3, sh3, Wf1[i].T)
        sc4, sh4 = _aff(st4, B * N, g_f[i], b_f[i])
        r2, st5 = _ffn2(r, sc3, sh3, Wf1[i].T, sc4, sh4, Wf2[i].T)
        sc, sh = _aff(st5, B * N, g2[i], b2[i])
        x_pre, act = r2, False

    return _head(x_pre, sc, sh, Ws1.T, g_s.reshape(1, HID),
                 b_s.reshape(1, HID), Ws2, bs2.reshape(NC, 1))


# speedup vs baseline: 13.6419x; 1.1544x over previous
import functools

import jax
import jax.numpy as jnp
from jax import lax
from jax.experimental import pallas as pl
from jax.experimental.pallas import tpu as pltpu
from jax.experimental.pallas import tpu_sc as plsc

B, N, DIM, DEPTH, K, HID, NC = 4, 2048, 64, 2, 16, 128, 13
EPS = 1e-5
RB = 256
NB = 256
NBLK = N // NB
NBK = NB * K


def _knn_body(p_ref, pt_ref, o_ref):
    p = p_ref[0]
    pt = pt_ref[0]
    sqr = jnp.sum(p * p, axis=1, keepdims=True)
    sqa = jnp.sum(pt * pt, axis=0, keepdims=True)
    dot = lax.dot_general(p.astype(jnp.bfloat16), pt.astype(jnp.bfloat16),
                          (((1,), (0,)), ((), ())),
                          preferred_element_type=jnp.float32)
    d = (sqr + sqa) - 2.0 * dot
    cols = lax.broadcasted_iota(jnp.int32, (RB, N), 1)
    kcols = lax.broadcasted_iota(jnp.int32, (RB, K), 1)
    kidx = jnp.zeros((RB, K), jnp.int32)
    for j in range(K):
        m = jnp.min(d, axis=1, keepdims=True)
        am = jnp.min(jnp.where(d == m, cols, N), axis=1, keepdims=True)
        kidx = jnp.where(kcols == j, am, kidx)
        d = jnp.where(cols == am, jnp.float32(jnp.inf), d)
    o_ref[0] = kidx + pl.program_id(0) * N


def _knn(points, pT):
    return pl.pallas_call(
        _knn_body,
        grid=(B, N // RB),
        in_specs=[
            pl.BlockSpec((1, RB, 3), lambda b, r: (b, r, 0)),
            pl.BlockSpec((1, 3, N), lambda b, r: (b, 0, 0)),
        ],
        out_specs=pl.BlockSpec((1, RB, K), lambda b, r: (b, r, 0)),
        out_shape=jax.ShapeDtypeStruct((B, N, K), jnp.int32),
    )(points, pT)


def _make_sc_gather(R, D, M):
    NW = 32
    Mw = M // NW
    CH = min(Mw, 65536 // D)
    NCH = Mw // CH
    mesh = plsc.VectorSubcoreMesh(core_axis_name="c", subcore_axis_name="s")

    @functools.partial(
        pl.kernel,
        mesh=mesh,
        compiler_params=pltpu.CompilerParams(use_tc_tiling_on_sc=False),
        out_type=jax.ShapeDtypeStruct((M, D), jnp.float32),
        scratch_types=[
            pltpu.VMEM((CH,), jnp.int32),
            pltpu.VMEM((CH, D), jnp.float32),
            pltpu.SemaphoreType.DMA,
        ],
    )
    def g(tab_hbm, idx_hbm, out_hbm, idx_v, rows_v, sem):
        wid = lax.axis_index("s") * 2 + lax.axis_index("c")
        base = wid * Mw
        for c in range(NCH):
            off = base + c * CH
            pltpu.sync_copy(idx_hbm.at[pl.ds(off, CH)], idx_v)
            pltpu.async_copy(tab_hbm.at[idx_v], rows_v, sem).wait()
            pltpu.sync_copy(rows_v, out_hbm.at[pl.ds(off, CH)])

    return g


def _acc_stats(acc_ref, st_ref, s, sq, nblk_last):
    first = (pl.program_id(0) == 0) & (pl.program_id(1) == 0)
    last = (pl.program_id(0) == B - 1) & (pl.program_id(1) == nblk_last)

    @pl.when(first)
    def _():
        acc_ref[...] = jnp.zeros_like(acc_ref)

    c = s.shape[-1]
    acc_ref[0:1] = acc_ref[0:1] + s.reshape(1, c)
    acc_ref[1:2] = acc_ref[1:2] + sq.reshape(1, c)

    @pl.when(last)
    def _():
        st_ref[...] = acc_ref[...]


def _embed_body(p_ref, w_ref, e_ref, st_ref, acc_ref):
    p = p_ref[0]
    w = w_ref[...]
    e = lax.dot_general(p.astype(jnp.bfloat16), w.astype(jnp.bfloat16),
                        (((1,), (0,)), ((), ())),
                        preferred_element_type=jnp.float32)
    e_ref[0] = e
    first = pl.program_id(0) == 0
    last = pl.program_id(0) == B - 1

    @pl.when(first)
    def _():
        acc_ref[...] = jnp.zeros_like(acc_ref)

    acc_ref[0:1] = acc_ref[0:1] + jnp.sum(e, axis=0).reshape(1, DIM)
    acc_ref[1:2] = acc_ref[1:2] + jnp.sum(e * e, axis=0).reshape(1, DIM)

    @pl.when(last)
    def _():
        st_ref[...] = acc_ref[...]


def _embed(points, WeT):
    return pl.pallas_call(
        _embed_body,
        grid=(B,),
        in_specs=[
            pl.BlockSpec((1, N, 3), lambda b: (b, 0, 0)),
            pl.BlockSpec((3, DIM), lambda b: (0, 0)),
        ],
        out_specs=[
            pl.BlockSpec((1, N, DIM), lambda b: (b, 0, 0)),
            pl.BlockSpec((8, DIM), lambda b: (0, 0)),
        ],
        out_shape=[
            jax.ShapeDtypeStruct((B, N, DIM), jnp.float32),
            jax.ShapeDtypeStruct((8, DIM), jnp.float32),
        ],
        scratch_shapes=[pltpu.VMEM((8, DIM), jnp.float32)],
    )(points, WeT)


def _bf(x):
    return x.astype(jnp.bfloat16)


def _mm(a, b):
    return lax.dot_general(_bf(a), _bf(b), (((a.ndim - 1,), (0,)), ((), ())),
                           preferred_element_type=jnp.float32)


def _qkv_body(act, xp_ref, sc_ref, sh_ref, wq_ref, wk_ref, wv_ref,
              x_ref, q_ref, kv_ref):
    x = xp_ref[0] * sc_ref[...] + sh_ref[...]
    if act:
        x = jnp.maximum(x, 0.0)
    x_ref[0] = x
    q_ref[0] = _mm(x, wq_ref[...])
    kv_ref[0, :, :DIM] = _mm(x, wk_ref[...])
    kv_ref[0, :, DIM:] = _mm(x, wv_ref[...])


def _qkv(x_pre, sc, sh, act, WqT, WkT, WvT):
    return pl.pallas_call(
        functools.partial(_qkv_body, act),
        grid=(B,),
        in_specs=[
            pl.BlockSpec((1, N, DIM), lambda b: (b, 0, 0)),
            pl.BlockSpec((1, DIM), lambda b: (0, 0)),
            pl.BlockSpec((1, DIM), lambda b: (0, 0)),
            pl.BlockSpec((DIM, DIM), lambda b: (0, 0)),
            pl.BlockSpec((DIM, DIM), lambda b: (0, 0)),
            pl.BlockSpec((DIM, DIM), lambda b: (0, 0)),
        ],
        out_specs=[
            pl.BlockSpec((1, N, DIM), lambda b: (b, 0, 0)),
            pl.BlockSpec((1, N, DIM), lambda b: (b, 0, 0)),
            pl.BlockSpec((1, N, 2 * DIM), lambda b: (b, 0, 0)),
        ],
        out_shape=[
            jax.ShapeDtypeStruct((B, N, DIM), jnp.float32),
            jax.ShapeDtypeStruct((B, N, DIM), jnp.float32),
            jax.ShapeDtypeStruct((B, N, 2 * DIM), jnp.float32),
        ],
    )(x_pre, sc, sh, WqT, WkT, WvT)


def _dot3(a, w):
    return lax.dot_general(_bf(a), _bf(w), (((2,), (0,)), ((), ())),
                           preferred_element_type=jnp.float32)


def _pd_body(p_ref, xn_ref, pd_ref, st_ref, acc_ref):
    p16 = p_ref[0]
    xn = xn_ref[0]
    pd = (p16[:, None, :] - xn.reshape(NB, K, 16)).reshape(NBK, 16)
    pd_ref[0] = pd
    s1 = jnp.sum(pd, axis=0)
    s2 = jnp.sum(pd * pd, axis=0)
    r1 = jnp.sum(pd * jnp.roll(pd, -1, axis=1), axis=0)
    r2 = jnp.sum(pd * jnp.roll(pd, -2, axis=1), axis=0)
    first = (pl.program_id(0) == 0) & (pl.program_id(1) == 0)
    last = (pl.program_id(0) == B - 1) & (pl.program_id(1) == NBLK - 1)

    @pl.when(first)
    def _():
        acc_ref[...] = jnp.zeros_like(acc_ref)

    acc_ref[0:1] = acc_ref[0:1] + s1.reshape(1, 16)
    acc_ref[1:2] = acc_ref[1:2] + s2.reshape(1, 16)
    acc_ref[2:3] = acc_ref[2:3] + r1.reshape(1, 16)
    acc_ref[3:4] = acc_ref[3:4] + r2.reshape(1, 16)

    @pl.when(last)
    def _():
        st_ref[...] = acc_ref[...]


def _pos_diff(points16, xyznn):
    return pl.pallas_call(
        _pd_body,
        grid=(B, NBLK),
        in_specs=[
            pl.BlockSpec((1, NB, 16), lambda b, r: (b, r, 0)),
            pl.BlockSpec((1, NBK, 16), lambda b, r: (b, r, 0)),
        ],
        out_specs=[
            pl.BlockSpec((1, NBK, 16), lambda b, r: (b, r, 0)),
            pl.BlockSpec((8, 16), lambda b, r: (0, 0)),
        ],
        out_shape=[
            jax.ShapeDtypeStruct((B, N * K, 16), jnp.float32),
            jax.ShapeDtypeStruct((8, 16), jnp.float32),
        ],
        scratch_shapes=[pltpu.VMEM((8, 16), jnp.float32)],
    )(points16, xyznn)


def _attn_pre_body(pd_ref, q_ref, kv_ref, w1_ref, sc1_ref, sh1_ref,
                   w2_ref, wa1_ref, pe_ref, t2_ref, st_ref, acc_ref):
    t1 = _mm(pd_ref[0], w1_ref[...])
    pe_in = jnp.maximum(t1 * sc1_ref[...] + sh1_ref[...], 0.0)
    pe = _mm(pe_in, w2_ref[...])
    pe_ref[0] = pe
    q3 = q_ref[0][:, None, :]
    k3 = kv_ref[0][:, :DIM].reshape(NB, K, DIM)
    a1 = (q3 - k3) + pe.reshape(NB, K, DIM)
    t2 = _dot3(a1, wa1_ref[...])
    t2_ref[0] = t2.reshape(NBK, DIM)
    s = jnp.sum(t2, axis=(0, 1))
    sq = jnp.sum(t2 * t2, axis=(0, 1))
    _acc_stats(acc_ref, st_ref, s, sq, NBLK - 1)


def _attn_pre(pd, q, kvnn, Wpe1T16, sc1, sh1, Wpe2T, Wam1T):
    return pl.pallas_call(
        _attn_pre_body,
        grid=(B, NBLK),
        in_specs=[
            pl.BlockSpec((1, NBK, 16), lambda b, r: (b, r, 0)),
            pl.BlockSpec((1, NB, DIM), lambda b, r: (b, r, 0)),
            pl.BlockSpec((1, NBK, 2 * DIM), lambda b, r: (b, r, 0)),
            pl.BlockSpec((16, DIM), lambda b, r: (0, 0)),
            pl.BlockSpec((1, DIM), lambda b, r: (0, 0)),
            pl.BlockSpec((1, DIM), lambda b, r: (0, 0)),
            pl.BlockSpec((DIM, DIM), lambda b, r: (0, 0)),
            pl.BlockSpec((DIM, DIM), lambda b, r: (0, 0)),
        ],
        out_specs=[
            pl.BlockSpec((1, NBK, DIM), lambda b, r: (b, r, 0)),
            pl.BlockSpec((1, NBK, DIM), lambda b, r: (b, r, 0)),
            pl.BlockSpec((8, DIM), lambda b, r: (0, 0)),
        ],
        out_shape=[
            jax.ShapeDtypeStruct((B, N * K, DIM), jnp.float32),
            jax.ShapeDtypeStruct((B, N * K, DIM), jnp.float32),
            jax.ShapeDtypeStruct((8, DIM), jnp.float32),
        ],
        scratch_shapes=[pltpu.VMEM((8, DIM), jnp.float32)],
    )(pd, q, kvnn, Wpe1T16, sc1, sh1, Wpe2T, Wam1T)


def _attn_out_body(pe_ref, t2_ref, kv_ref, x_ref, sc2_ref, sh2_ref,
                   wa2_ref, r_ref, st_ref, acc_ref):
    t2 = t2_ref[0].reshape(NB, K, DIM)
    am_in = jnp.maximum(t2 * sc2_ref[...][None] + sh2_ref[...][None], 0.0)
    al = _dot3(am_in, wa2_ref[...])
    m = jnp.max(al, axis=1, keepdims=True)
    e = jnp.exp(al - m)
    attn = e / jnp.sum(e, axis=1, keepdims=True)
    v3 = kv_ref[0][:, DIM:].reshape(NB, K, DIM)
    pe3 = pe_ref[0].reshape(NB, K, DIM)
    out = jnp.sum(attn * (v3 + pe3), axis=1)
    r = x_ref[0] + out
    r_ref[0] = r
    s = jnp.sum(r, axis=0)
    sq = jnp.sum(r * r, axis=0)
    _acc_stats(acc_ref, st_ref, s, sq, NBLK - 1)


def _attn_out(pe, t2, kvnn, x, sc2, sh2, Wam2T):
    return pl.pallas_call(
        _attn_out_body,
        grid=(B, NBLK),
        in_specs=[
            pl.BlockSpec((1, NBK, DIM), lambda b, r: (b, r, 0)),
            pl.BlockSpec((1, NBK, DIM), lambda b, r: (b, r, 0)),
            pl.BlockSpec((1, NBK, 2 * DIM), lambda b, r: (b, r, 0)),
            pl.BlockSpec((1, NB, DIM), lambda b, r: (b, r, 0)),
            pl.BlockSpec((1, DIM), lambda b, r: (0, 0)),
            pl.BlockSpec((1, DIM), lambda b, r: (0, 0)),
            pl.BlockSpec((DIM, DIM), lambda b, r: (0, 0)),
        ],
        out_specs=[
            pl.BlockSpec((1, NB, DIM), lambda b, r: (b, r, 0)),
            pl.BlockSpec((8, DIM), lambda b, r: (0, 0)),
        ],
        out_shape=[
            jax.ShapeDtypeStruct((B, N, DIM), jnp.float32),
            jax.ShapeDtypeStruct((8, DIM), jnp.float32),
        ],
        scratch_shapes=[pltpu.VMEM((8, DIM), jnp.float32)],
    )(pe, t2, kvnn, x, sc2, sh2, Wam2T)


def _ffn1_body(r_ref, sc_ref, sh_ref, wf1_ref, st_ref, acc_ref):
    x1 = r_ref[0] * sc_ref[...] + sh_ref[...]
    t3 = _mm(x1, wf1_ref[...])
    first = pl.program_id(0) == 0
    last = pl.program_id(0) == B - 1

    @pl.when(first)
    def _():
        acc_ref[...] = jnp.zeros_like(acc_ref)

    acc_ref[0:1] = acc_ref[0:1] + jnp.sum(t3, axis=0).reshape(1, HID)
    acc_ref[1:2] = acc_ref[1:2] + jnp.sum(t3 * t3, axis=0).reshape(1, HID)

    @pl.when(last)
    def _():
        st_ref[...] = acc_ref[...]


def _ffn1(r, sc3, sh3, Wf1T):
    return pl.pallas_call(
        _ffn1_body,
        grid=(B,),
        in_specs=[
            pl.BlockSpec((1, N, DIM), lambda b: (b, 0, 0)),
            pl.BlockSpec((1, DIM), lambda b: (0, 0)),
            pl.BlockSpec((1, DIM), lambda b: (0, 0)),
            pl.BlockSpec((DIM, HID), lambda b: (0, 0)),
        ],
        out_specs=pl.BlockSpec((8, HID), lambda b: (0, 0)),
        out_shape=jax.ShapeDtypeStruct((8, HID), jnp.float32),
        scratch_shapes=[pltpu.VMEM((8, HID), jnp.float32)],
    )(r, sc3, sh3, Wf1T)


def _ffn2_body(r_ref, sc3_ref, sh3_ref, wf1_ref, sc4_ref, sh4_ref, wf2_ref,
               r2_ref, st_ref, acc_ref):
    x1 = r_ref[0] * sc3_ref[...] + sh3_ref[...]
    t3 = _mm(x1, wf1_ref[...])
    h = jnp.maximum(t3 * sc4_ref[...] + sh4_ref[...], 0.0)
    t4 = _mm(h, wf2_ref[...])
    r2 = x1 + t4
    r2_ref[0] = r2
    first = pl.program_id(0) == 0
    last = pl.program_id(0) == B - 1

    @pl.when(first)
    def _():
        acc_ref[...] = jnp.zeros_like(acc_ref)

    acc_ref[0:1] = acc_ref[0:1] + jnp.sum(r2, axis=0).reshape(1, DIM)
    acc_ref[1:2] = acc_ref[1:2] + jnp.sum(r2 * r2, axis=0).reshape(1, DIM)

    @pl.when(last)
    def _():
        st_ref[...] = acc_ref[...]


def _ffn2(r, sc3, sh3, Wf1T, sc4, sh4, Wf2T):
    return pl.pallas_call(
        _ffn2_body,
        grid=(B,),
        in_specs=[
            pl.BlockSpec((1, N, DIM), lambda b: (b, 0, 0)),
            pl.BlockSpec((1, DIM), lambda b: (0, 0)),
            pl.BlockSpec((1, DIM), lambda b: (0, 0)),
            pl.BlockSpec((DIM, HID), lambda b: (0, 0)),
            pl.BlockSpec((1, HID), lambda b: (0, 0)),
            pl.BlockSpec((1, HID), lambda b: (0, 0)),
            pl.BlockSpec((HID, DIM), lambda b: (0, 0)),
        ],
        out_specs=[
            pl.BlockSpec((1, N, DIM), lambda b: (b, 0, 0)),
            pl.BlockSpec((8, DIM), lambda b: (0, 0)),
        ],
        out_shape=[
            jax.ShapeDtypeStruct((B, N, DIM), jnp.float32),
            jax.ShapeDtypeStruct((8, DIM), jnp.float32),
        ],
        scratch_shapes=[pltpu.VMEM((8, DIM), jnp.float32)],
    )(r, sc3, sh3, Wf1T, sc4, sh4, Wf2T)


def _head_body(r2_ref, sc_ref, sh_ref, ws1_ref, gs_ref, bs_ref, ws2_ref,
               bs2_ref, o_ref):
    ws1 = ws1_ref[...]
    hs = []
    s = jnp.zeros((1, HID), jnp.float32)
    sq = jnp.zeros((1, HID), jnp.float32)
    for b in range(B):
        x = r2_ref[b] * sc_ref[...] + sh_ref[...]
        h = _mm(x, ws1)
        hs.append(h)
        s = s + jnp.sum(h, axis=0).reshape(1, HID)
        sq = sq + jnp.sum(h * h, axis=0).reshape(1, HID)
    cnt = float(B * N)
    m = s / cnt
    v = sq / cnt - m * m
    scale = gs_ref[...] / jnp.sqrt(v + EPS)
    shift = bs_ref[...] - m * scale
    ws2 = ws2_ref[...]
    for b in range(B):
        h = jnp.maximum(hs[b] * scale + shift, 0.0)
        o = lax.dot_general(_bf(ws2), _bf(h), (((1,), (1,)), ((), ())),
                            preferred_element_type=jnp.float32)
        o_ref[b] = o + bs2_ref[...]


def _head(r2, sc5, sh5, Ws1T, g_s, b_s, Ws2, bs2c):
    return pl.pallas_call(
        _head_body,
        out_shape=jax.ShapeDtypeStruct((B, NC, N), jnp.float32),
    )(r2, sc5, sh5, Ws1T, g_s, b_s, Ws2, bs2c)


def _aff(st, cnt, g, b):
    s, sq = st[0, :g.shape[0]], st[1, :g.shape[0]]
    m = s / cnt
    v = sq / cnt - m * m
    sc = g / jnp.sqrt(v + EPS)
    sh = b - m * sc
    return sc.reshape(1, -1), sh.reshape(1, -1)


def kernel(points, We, g_e, b_e, Wq, Wk, Wv, Wpe1, g_pe, b_pe, Wpe2, Wam1,
           g_am, b_am, Wam2, g1, b1, Wf1, g_f, b_f, Wf2, g2, b2, Ws1, g_s,
           b_s, Ws2, bs2):
    pT = jnp.transpose(points, (0, 2, 1))
    idx = _knn(points, pT)
    idxf = idx.reshape(B * N * K)

    xyztab = jnp.pad(points.reshape(B * N, 3), ((0, 0), (0, 13)))
    xyznn = _make_sc_gather(B * N, 16, B * N * K)(xyztab, idxf)
    xyznn = xyznn.reshape(B, N * K, 16)

    pd, st_pd = _pos_diff(xyztab.reshape(B, N, 16), xyznn)
    cnt_pd = float(B * N * K)
    mu3 = st_pd[0, :3] / cnt_pd
    d0, d1, d2_ = st_pd[1, 0], st_pd[1, 1], st_pd[1, 2]
    xy, yz, xz = st_pd[2, 0], st_pd[2, 1], st_pd[3, 0]
    Smat = jnp.stack([
        jnp.stack([d0, xy, xz]),
        jnp.stack([xy, d1, yz]),
        jnp.stack([xz, yz, d2_]),
    ])

    e_pre, st_e = _embed(points, We.T)
    sc, sh = _aff(st_e, B * N, g_e, b_e)
    x_pre, act = e_pre, True

    kv_gather = _make_sc_gather(B * N, 2 * DIM, B * N * K)
    for i in range(DEPTH):
        x, q, kv = _qkv(x_pre, sc, sh, act, Wq[i].T, Wk[i].T, Wv[i].T)
        kvnn = kv_gather(kv.reshape(B * N, 2 * DIM), idxf)
        kvnn = kvnn.reshape(B, N * K, 2 * DIM)
        w1T, w2T, wa1T, wa2T = Wpe1[i].T, Wpe2[i].T, Wam1[i].T, Wam2[i].T
        w1T16 = jnp.pad(w1T, ((0, 13), (0, 0)))
        m1 = mu3 @ w1T
        e2 = jnp.sum(w1T * (Smat @ w1T), axis=0) / cnt_pd
        v1 = e2 - m1 * m1
        sc1 = (g_pe[i] / jnp.sqrt(v1 + EPS)).reshape(1, DIM)
        sh1 = (b_pe[i] - m1 * sc1[0]).reshape(1, DIM)
        pe, t2, st2 = _attn_pre(pd, q, kvnn, w1T16, sc1, sh1, w2T, wa1T)
        sc2, sh2 = _aff(st2, B * N * K, g_am[i], b_am[i])
        r, st3 = _attn_out(pe, t2, kvnn, x, sc2, sh2, wa2T)
        sc3, sh3 = _aff(st3, B * N, g1[i], b1[i])
        st4 = _ffn1(r, sc3, sh3, Wf1[i].T)
        sc4, sh4 = _aff(st4, B * N, g_f[i], b_f[i])
        r2, st5 = _ffn2(r, sc3, sh3, Wf1[i].T, sc4, sh4, Wf2[i].T)
        sc, sh = _aff(st5, B * N, g2[i], b2[i])
        x_pre, act = r2, False

    return _head(x_pre, sc, sh, Ws1.T, g_s.reshape(1, HID),
                 b_s.reshape(1, HID), Ws2, bs2.reshape(NC, 1))


# speedup vs baseline: 14.0883x; 1.0327x over previous
import functools

import jax
import jax.numpy as jnp
from jax import lax
from jax.experimental import pallas as pl
from jax.experimental.pallas import tpu as pltpu
from jax.experimental.pallas import tpu_sc as plsc

B, N, DIM, DEPTH, K, HID, NC = 4, 2048, 64, 2, 16, 128, 13
EPS = 1e-5
RB = 256
NB = 256
NBLK = N // NB
NBK = NB * K


def _knn_body(p_ref, pt_ref, o_ref):
    p = p_ref[0]
    pt = pt_ref[0]
    sqr = jnp.sum(p * p, axis=1, keepdims=True)
    sqa = jnp.sum(pt * pt, axis=0, keepdims=True)
    dot = lax.dot_general(p.astype(jnp.bfloat16), pt.astype(jnp.bfloat16),
                          (((1,), (0,)), ((), ())),
                          preferred_element_type=jnp.float32)
    d = (sqr + sqa) - 2.0 * dot
    H = N // 2
    left, right = d[:, :H], d[:, H:]
    lt = left <= right
    dlo = jnp.minimum(left, right)
    dhi = jnp.maximum(left, right)
    colsh = lax.broadcasted_iota(jnp.int32, (RB, H), 1)
    ilo = jnp.where(lt, colsh, colsh + H)
    ihi = jnp.where(lt, colsh + H, colsh)
    kcols = lax.broadcasted_iota(jnp.int32, (RB, K), 1)
    kidx = jnp.zeros((RB, K), jnp.int32)
    for j in range(K):
        m = jnp.min(dlo, axis=1, keepdims=True)
        am = jnp.min(jnp.where(dlo == m, ilo, N), axis=1, keepdims=True)
        kidx = jnp.where(kcols == j, am, kidx)
        pm = ilo == am
        dlo = jnp.where(pm, dhi, dlo)
        ilo = jnp.where(pm, ihi, ilo)
        dhi = jnp.where(pm, jnp.float32(jnp.inf), dhi)
    o_ref[0] = kidx + pl.program_id(0) * N


def _knn(points, pT):
    return pl.pallas_call(
        _knn_body,
        grid=(B, N // RB),
        in_specs=[
            pl.BlockSpec((1, RB, 3), lambda b, r: (b, r, 0)),
            pl.BlockSpec((1, 3, N), lambda b, r: (b, 0, 0)),
        ],
        out_specs=pl.BlockSpec((1, RB, K), lambda b, r: (b, r, 0)),
        out_shape=jax.ShapeDtypeStruct((B, N, K), jnp.int32),
    )(points, pT)


def _make_sc_gather(R, D, M):
    NW = 32
    Mw = M // NW
    CH = min(Mw, 65536 // D)
    NCH = Mw // CH
    mesh = plsc.VectorSubcoreMesh(core_axis_name="c", subcore_axis_name="s")

    @functools.partial(
        pl.kernel,
        mesh=mesh,
        compiler_params=pltpu.CompilerParams(use_tc_tiling_on_sc=False),
        out_type=jax.ShapeDtypeStruct((M, D), jnp.float32),
        scratch_types=[
            pltpu.VMEM((CH,), jnp.int32),
            pltpu.VMEM((CH, D), jnp.float32),
            pltpu.SemaphoreType.DMA,
        ],
    )
    def g(tab_hbm, idx_hbm, out_hbm, idx_v, rows_v, sem):
        wid = lax.axis_index("s") * 2 + lax.axis_index("c")
        base = wid * Mw
        for c in range(NCH):
            off = base + c * CH
            pltpu.sync_copy(idx_hbm.at[pl.ds(off, CH)], idx_v)
            pltpu.async_copy(tab_hbm.at[idx_v], rows_v, sem).wait()
            pltpu.sync_copy(rows_v, out_hbm.at[pl.ds(off, CH)])

    return g


def _acc_stats(acc_ref, st_ref, s, sq, nblk_last):
    first = (pl.program_id(0) == 0) & (pl.program_id(1) == 0)
    last = (pl.program_id(0) == B - 1) & (pl.program_id(1) == nblk_last)

    @pl.when(first)
    def _():
        acc_ref[...] = jnp.zeros_like(acc_ref)

    c = s.shape[-1]
    acc_ref[0:1] = acc_ref[0:1] + s.reshape(1, c)
    acc_ref[1:2] = acc_ref[1:2] + sq.reshape(1, c)

    @pl.when(last)
    def _():
        st_ref[...] = acc_ref[...]


def _embed_body(p_ref, w_ref, e_ref, st_ref, acc_ref):
    p = p_ref[0]
    w = w_ref[...]
    e = lax.dot_general(p.astype(jnp.bfloat16), w.astype(jnp.bfloat16),
                        (((1,), (0,)), ((), ())),
                        preferred_element_type=jnp.float32)
    e_ref[0] = e
    first = pl.program_id(0) == 0
    last = pl.program_id(0) == B - 1

    @pl.when(first)
    def _():
        acc_ref[...] = jnp.zeros_like(acc_ref)

    acc_ref[0:1] = acc_ref[0:1] + jnp.sum(e, axis=0).reshape(1, DIM)
    acc_ref[1:2] = acc_ref[1:2] + jnp.sum(e * e, axis=0).reshape(1, DIM)

    @pl.when(last)
    def _():
        st_ref[...] = acc_ref[...]


def _embed(points, WeT):
    return pl.pallas_call(
        _embed_body,
        grid=(B,),
        in_specs=[
            pl.BlockSpec((1, N, 3), lambda b: (b, 0, 0)),
            pl.BlockSpec((3, DIM), lambda b: (0, 0)),
        ],
        out_specs=[
            pl.BlockSpec((1, N, DIM), lambda b: (b, 0, 0)),
            pl.BlockSpec((8, DIM), lambda b: (0, 0)),
        ],
        out_shape=[
            jax.ShapeDtypeStruct((B, N, DIM), jnp.float32),
            jax.ShapeDtypeStruct((8, DIM), jnp.float32),
        ],
        scratch_shapes=[pltpu.VMEM((8, DIM), jnp.float32)],
    )(points, WeT)


def _bf(x):
    return x.astype(jnp.bfloat16)


def _mm(a, b):
    return lax.dot_general(_bf(a), _bf(b), (((a.ndim - 1,), (0,)), ((), ())),
                           preferred_element_type=jnp.float32)


def _qkv_body(act, xp_ref, sc_ref, sh_ref, wq_ref, wk_ref, wv_ref,
              x_ref, q_ref, kv_ref):
    x = xp_ref[0] * sc_ref[...] + sh_ref[...]
    if act:
        x = jnp.maximum(x, 0.0)
    x_ref[0] = x
    q_ref[0] = _mm(x, wq_ref[...])
    kv_ref[0, :, :DIM] = _mm(x, wk_ref[...])
    kv_ref[0, :, DIM:] = _mm(x, wv_ref[...])


def _qkv(x_pre, sc, sh, act, WqT, WkT, WvT):
    return pl.pallas_call(
        functools.partial(_qkv_body, act),
        grid=(B,),
        in_specs=[
            pl.BlockSpec((1, N, DIM), lambda b: (b, 0, 0)),
            pl.BlockSpec((1, DIM), lambda b: (0, 0)),
            pl.BlockSpec((1, DIM), lambda b: (0, 0)),
            pl.BlockSpec((DIM, DIM), lambda b: (0, 0)),
            pl.BlockSpec((DIM, DIM), lambda b: (0, 0)),
            pl.BlockSpec((DIM, DIM), lambda b: (0, 0)),
        ],
        out_specs=[
            pl.BlockSpec((1, N, DIM), lambda b: (b, 0, 0)),
            pl.BlockSpec((1, N, DIM), lambda b: (b, 0, 0)),
            pl.BlockSpec((1, N, 2 * DIM), lambda b: (b, 0, 0)),
        ],
        out_shape=[
            jax.ShapeDtypeStruct((B, N, DIM), jnp.float32),
            jax.ShapeDtypeStruct((B, N, DIM), jnp.float32),
            jax.ShapeDtypeStruct((B, N, 2 * DIM), jnp.float32),
        ],
    )(x_pre, sc, sh, WqT, WkT, WvT)


def _dot3(a, w):
    return lax.dot_general(_bf(a), _bf(w), (((2,), (0,)), ((), ())),
                           preferred_element_type=jnp.float32)


def _pd_body(p_ref, xn_ref, pd_ref, st_ref, acc_ref):
    p16 = p_ref[0]
    xn = xn_ref[0]
    pd = (p16[:, None, :] - xn.reshape(NB, K, 16)).reshape(NBK, 16)
    pd_ref[0] = pd
    s1 = jnp.sum(pd, axis=0)
    s2 = jnp.sum(pd * pd, axis=0)
    r1 = jnp.sum(pd * jnp.roll(pd, -1, axis=1), axis=0)
    r2 = jnp.sum(pd * jnp.roll(pd, -2, axis=1), axis=0)
    first = (pl.program_id(0) == 0) & (pl.program_id(1) == 0)
    last = (pl.program_id(0) == B - 1) & (pl.program_id(1) == NBLK - 1)

    @pl.when(first)
    def _():
        acc_ref[...] = jnp.zeros_like(acc_ref)

    acc_ref[0:1] = acc_ref[0:1] + s1.reshape(1, 16)
    acc_ref[1:2] = acc_ref[1:2] + s2.reshape(1, 16)
    acc_ref[2:3] = acc_ref[2:3] + r1.reshape(1, 16)
    acc_ref[3:4] = acc_ref[3:4] + r2.reshape(1, 16)

    @pl.when(last)
    def _():
        st_ref[...] = acc_ref[...]


def _pos_diff(points16, xyznn):
    return pl.pallas_call(
        _pd_body,
        grid=(B, NBLK),
        in_specs=[
            pl.BlockSpec((1, NB, 16), lambda b, r: (b, r, 0)),
            pl.BlockSpec((1, NBK, 16), lambda b, r: (b, r, 0)),
        ],
        out_specs=[
            pl.BlockSpec((1, NBK, 16), lambda b, r: (b, r, 0)),
            pl.BlockSpec((8, 16), lambda b, r: (0, 0)),
        ],
        out_shape=[
            jax.ShapeDtypeStruct((B, N * K, 16), jnp.float32),
            jax.ShapeDtypeStruct((8, 16), jnp.float32),
        ],
        scratch_shapes=[pltpu.VMEM((8, 16), jnp.float32)],
    )(points16, xyznn)


def _attn_pre_body(pd_ref, q_ref, kv_ref, w1_ref, sc1_ref, sh1_ref,
                   w2_ref, wa1_ref, pe_ref, t2_ref, st_ref, acc_ref):
    t1 = _mm(pd_ref[0], w1_ref[...])
    pe_in = jnp.maximum(t1 * sc1_ref[...] + sh1_ref[...], 0.0)
    pe = _mm(pe_in, w2_ref[...])
    pe_ref[0] = pe
    q3 = q_ref[0][:, None, :]
    k3 = kv_ref[0][:, :DIM].reshape(NB, K, DIM)
    a1 = (q3 - k3) + pe.reshape(NB, K, DIM)
    t2 = _dot3(a1, wa1_ref[...])
    t2_ref[0] = t2.reshape(NBK, DIM)
    s = jnp.sum(t2, axis=(0, 1))
    sq = jnp.sum(t2 * t2, axis=(0, 1))
    _acc_stats(acc_ref, st_ref, s, sq, NBLK - 1)


def _attn_pre(pd, q, kvnn, Wpe1T16, sc1, sh1, Wpe2T, Wam1T):
    return pl.pallas_call(
        _attn_pre_body,
        grid=(B, NBLK),
        in_specs=[
            pl.BlockSpec((1, NBK, 16), lambda b, r: (b, r, 0)),
            pl.BlockSpec((1, NB, DIM), lambda b, r: (b, r, 0)),
            pl.BlockSpec((1, NBK, 2 * DIM), lambda b, r: (b, r, 0)),
            pl.BlockSpec((16, DIM), lambda b, r: (0, 0)),
            pl.BlockSpec((1, DIM), lambda b, r: (0, 0)),
            pl.BlockSpec((1, DIM), lambda b, r: (0, 0)),
            pl.BlockSpec((DIM, DIM), lambda b, r: (0, 0)),
            pl.BlockSpec((DIM, DIM), lambda b, r: (0, 0)),
        ],
        out_specs=[
            pl.BlockSpec((1, NBK, DIM), lambda b, r: (b, r, 0)),
            pl.BlockSpec((1, NBK, DIM), lambda b, r: (b, r, 0)),
            pl.BlockSpec((8, DIM), lambda b, r: (0, 0)),
        ],
        out_shape=[
            jax.ShapeDtypeStruct((B, N * K, DIM), jnp.float32),
            jax.ShapeDtypeStruct((B, N * K, DIM), jnp.float32),
            jax.ShapeDtypeStruct((8, DIM), jnp.float32),
        ],
        scratch_shapes=[pltpu.VMEM((8, DIM), jnp.float32)],
    )(pd, q, kvnn, Wpe1T16, sc1, sh1, Wpe2T, Wam1T)


def _attn_out_body(pe_ref, t2_ref, kv_ref, x_ref, sc2_ref, sh2_ref,
                   wa2_ref, r_ref, st_ref, acc_ref):
    t2 = t2_ref[0].reshape(NB, K, DIM)
    am_in = jnp.maximum(t2 * sc2_ref[...][None] + sh2_ref[...][None], 0.0)
    al = _dot3(am_in, wa2_ref[...])
    m = jnp.max(al, axis=1, keepdims=True)
    e = jnp.exp(al - m)
    attn = e / jnp.sum(e, axis=1, keepdims=True)
    v3 = kv_ref[0][:, DIM:].reshape(NB, K, DIM)
    pe3 = pe_ref[0].reshape(NB, K, DIM)
    out = jnp.sum(attn * (v3 + pe3), axis=1)
    r = x_ref[0] + out
    r_ref[0] = r
    s = jnp.sum(r, axis=0)
    sq = jnp.sum(r * r, axis=0)
    _acc_stats(acc_ref, st_ref, s, sq, NBLK - 1)


def _attn_out(pe, t2, kvnn, x, sc2, sh2, Wam2T):
    return pl.pallas_call(
        _attn_out_body,
        grid=(B, NBLK),
        in_specs=[
            pl.BlockSpec((1, NBK, DIM), lambda b, r: (b, r, 0)),
            pl.BlockSpec((1, NBK, DIM), lambda b, r: (b, r, 0)),
            pl.BlockSpec((1, NBK, 2 * DIM), lambda b, r: (b, r, 0)),
            pl.BlockSpec((1, NB, DIM), lambda b, r: (b, r, 0)),
            pl.BlockSpec((1, DIM), lambda b, r: (0, 0)),
            pl.BlockSpec((1, DIM), lambda b, r: (0, 0)),
            pl.BlockSpec((DIM, DIM), lambda b, r: (0, 0)),
        ],
        out_specs=[
            pl.BlockSpec((1, NB, DIM), lambda b, r: (b, r, 0)),
            pl.BlockSpec((8, DIM), lambda b, r: (0, 0)),
        ],
        out_shape=[
            jax.ShapeDtypeStruct((B, N, DIM), jnp.float32),
            jax.ShapeDtypeStruct((8, DIM), jnp.float32),
        ],
        scratch_shapes=[pltpu.VMEM((8, DIM), jnp.float32)],
    )(pe, t2, kvnn, x, sc2, sh2, Wam2T)


def _ffn1_body(r_ref, sc_ref, sh_ref, wf1_ref, st_ref, acc_ref):
    x1 = r_ref[0] * sc_ref[...] + sh_ref[...]
    t3 = _mm(x1, wf1_ref[...])
    first = pl.program_id(0) == 0
    last = pl.program_id(0) == B - 1

    @pl.when(first)
    def _():
        acc_ref[...] = jnp.zeros_like(acc_ref)

    acc_ref[0:1] = acc_ref[0:1] + jnp.sum(t3, axis=0).reshape(1, HID)
    acc_ref[1:2] = acc_ref[1:2] + jnp.sum(t3 * t3, axis=0).reshape(1, HID)

    @pl.when(last)
    def _():
        st_ref[...] = acc_ref[...]


def _ffn1(r, sc3, sh3, Wf1T):
    return pl.pallas_call(
        _ffn1_body,
        grid=(B,),
        in_specs=[
            pl.BlockSpec((1, N, DIM), lambda b: (b, 0, 0)),
            pl.BlockSpec((1, DIM), lambda b: (0, 0)),
            pl.BlockSpec((1, DIM), lambda b: (0, 0)),
            pl.BlockSpec((DIM, HID), lambda b: (0, 0)),
        ],
        out_specs=pl.BlockSpec((8, HID), lambda b: (0, 0)),
        out_shape=jax.ShapeDtypeStruct((8, HID), jnp.float32),
        scratch_shapes=[pltpu.VMEM((8, HID), jnp.float32)],
    )(r, sc3, sh3, Wf1T)


def _ffn2_body(r_ref, sc3_ref, sh3_ref, wf1_ref, sc4_ref, sh4_ref, wf2_ref,
               r2_ref, st_ref, acc_ref):
    x1 = r_ref[0] * sc3_ref[...] + sh3_ref[...]
    t3 = _mm(x1, wf1_ref[...])
    h = jnp.maximum(t3 * sc4_ref[...] + sh4_ref[...], 0.0)
    t4 = _mm(h, wf2_ref[...])
    r2 = x1 + t4
    r2_ref[0] = r2
    first = pl.program_id(0) == 0
    last = pl.program_id(0) == B - 1

    @pl.when(first)
    def _():
        acc_ref[...] = jnp.zeros_like(acc_ref)

    acc_ref[0:1] = acc_ref[0:1] + jnp.sum(r2, axis=0).reshape(1, DIM)
    acc_ref[1:2] = acc_ref[1:2] + jnp.sum(r2 * r2, axis=0).reshape(1, DIM)

    @pl.when(last)
    def _():
        st_ref[...] = acc_ref[...]


def _ffn2(r, sc3, sh3, Wf1T, sc4, sh4, Wf2T):
    return pl.pallas_call(
        _ffn2_body,
        grid=(B,),
        in_specs=[
            pl.BlockSpec((1, N, DIM), lambda b: (b, 0, 0)),
            pl.BlockSpec((1, DIM), lambda b: (0, 0)),
            pl.BlockSpec((1, DIM), lambda b: (0, 0)),
            pl.BlockSpec((DIM, HID), lambda b: (0, 0)),
            pl.BlockSpec((1, HID), lambda b: (0, 0)),
            pl.BlockSpec((1, HID), lambda b: (0, 0)),
            pl.BlockSpec((HID, DIM), lambda b: (0, 0)),
        ],
        out_specs=[
            pl.BlockSpec((1, N, DIM), lambda b: (b, 0, 0)),
            pl.BlockSpec((8, DIM), lambda b: (0, 0)),
        ],
        out_shape=[
            jax.ShapeDtypeStruct((B, N, DIM), jnp.float32),
            jax.ShapeDtypeStruct((8, DIM), jnp.float32),
        ],
        scratch_shapes=[pltpu.VMEM((8, DIM), jnp.float32)],
    )(r, sc3, sh3, Wf1T, sc4, sh4, Wf2T)


def _head_body(r2_ref, sc_ref, sh_ref, ws1_ref, gs_ref, bs_ref, ws2_ref,
               bs2_ref, o_ref):
    ws1 = ws1_ref[...]
    hs = []
    s = jnp.zeros((1, HID), jnp.float32)
    sq = jnp.zeros((1, HID), jnp.float32)
    for b in range(B):
        x = r2_ref[b] * sc_ref[...] + sh_ref[...]
        h = _mm(x, ws1)
        hs.append(h)
        s = s + jnp.sum(h, axis=0).reshape(1, HID)
        sq = sq + jnp.sum(h * h, axis=0).reshape(1, HID)
    cnt = float(B * N)
    m = s / cnt
    v = sq / cnt - m * m
    scale = gs_ref[...] / jnp.sqrt(v + EPS)
    shift = bs_ref[...] - m * scale
    ws2 = ws2_ref[...]
    for b in range(B):
        h = jnp.maximum(hs[b] * scale + shift, 0.0)
        o = lax.dot_general(_bf(ws2), _bf(h), (((1,), (1,)), ((), ())),
                            preferred_element_type=jnp.float32)
        o_ref[b] = o + bs2_ref[...]


def _head(r2, sc5, sh5, Ws1T, g_s, b_s, Ws2, bs2c):
    return pl.pallas_call(
        _head_body,
        out_shape=jax.ShapeDtypeStruct((B, NC, N), jnp.float32),
    )(r2, sc5, sh5, Ws1T, g_s, b_s, Ws2, bs2c)


def _aff(st, cnt, g, b):
    s, sq = st[0, :g.shape[0]], st[1, :g.shape[0]]
    m = s / cnt
    v = sq / cnt - m * m
    sc = g / jnp.sqrt(v + EPS)
    sh = b - m * sc
    return sc.reshape(1, -1), sh.reshape(1, -1)


def kernel(points, We, g_e, b_e, Wq, Wk, Wv, Wpe1, g_pe, b_pe, Wpe2, Wam1,
           g_am, b_am, Wam2, g1, b1, Wf1, g_f, b_f, Wf2, g2, b2, Ws1, g_s,
           b_s, Ws2, bs2):
    pT = jnp.transpose(points, (0, 2, 1))
    idx = _knn(points, pT)
    idxf = idx.reshape(B * N * K)

    xyztab = jnp.pad(points.reshape(B * N, 3), ((0, 0), (0, 13)))
    xyznn = _make_sc_gather(B * N, 16, B * N * K)(xyztab, idxf)
    xyznn = xyznn.reshape(B, N * K, 16)

    pd, st_pd = _pos_diff(xyztab.reshape(B, N, 16), xyznn)
    cnt_pd = float(B * N * K)
    mu3 = st_pd[0, :3] / cnt_pd
    d0, d1, d2_ = st_pd[1, 0], st_pd[1, 1], st_pd[1, 2]
    xy, yz, xz = st_pd[2, 0], st_pd[2, 1], st_pd[3, 0]
    Smat = jnp.stack([
        jnp.stack([d0, xy, xz]),
        jnp.stack([xy, d1, yz]),
        jnp.stack([xz, yz, d2_]),
    ])

    e_pre, st_e = _embed(points, We.T)
    sc, sh = _aff(st_e, B * N, g_e, b_e)
    x_pre, act = e_pre, True

    kv_gather = _make_sc_gather(B * N, 2 * DIM, B * N * K)
    for i in range(DEPTH):
        x, q, kv = _qkv(x_pre, sc, sh, act, Wq[i].T, Wk[i].T, Wv[i].T)
        kvnn = kv_gather(kv.reshape(B * N, 2 * DIM), idxf)
        kvnn = kvnn.reshape(B, N * K, 2 * DIM)
        w1T, w2T, wa1T, wa2T = Wpe1[i].T, Wpe2[i].T, Wam1[i].T, Wam2[i].T
        w1T16 = jnp.pad(w1T, ((0, 13), (0, 0)))
        m1 = mu3 @ w1T
        e2 = jnp.sum(w1T * (Smat @ w1T), axis=0) / cnt_pd
        v1 = e2 - m1 * m1
        sc1 = (g_pe[i] / jnp.sqrt(v1 + EPS)).reshape(1, DIM)
        sh1 = (b_pe[i] - m1 * sc1[0]).reshape(1, DIM)
        pe, t2, st2 = _attn_pre(pd, q, kvnn, w1T16, sc1, sh1, w2T, wa1T)
        sc2, sh2 = _aff(st2, B * N * K, g_am[i], b_am[i])
        r, st3 = _attn_out(pe, t2, kvnn, x, sc2, sh2, wa2T)
        sc3, sh3 = _aff(st3, B * N, g1[i], b1[i])
        st4 = _ffn1(r, sc3, sh3, Wf1[i].T)
        sc4, sh4 = _aff(st4, B * N, g_f[i], b_f[i])
        r2, st5 = _ffn2(r, sc3, sh3, Wf1[i].T, sc4, sh4, Wf2[i].T)
        sc, sh = _aff(st5, B * N, g2[i], b2[i])
        x_pre, act = r2, False

    return _head(x_pre, sc, sh, Ws1.T, g_s.reshape(1, HID),
                 b_s.reshape(1, HID), Ws2, bs2.reshape(NC, 1))
